# baseline (speedup 1.0000x reference)
import functools

import jax
import jax.numpy as jnp
from jax import lax
from jax.experimental import pallas as pl
from jax.experimental.pallas import tpu as pltpu
from jax.experimental.pallas import tpu_sc as plsc

B, S, H = 64, 512, 768
V = 30522
EPS = 1e-12

NC, NS = 2, 16
NW = NC * NS
TOK = B * S
CHUNK = 64

PIPE_SEQS = (4, 20, 24, 16)
assert sum(PIPE_SEQS) == B

SEQ_BLK = 4


@functools.lru_cache(maxsize=None)
def _sc_gather_fn(nseq):
  tok = nseq * S
  tpw = tok // NW
  chunk = min(CHUNK, tpw)
  nchunk = tpw // chunk
  mesh = plsc.VectorSubcoreMesh(core_axis_name="c", subcore_axis_name="s",
                                num_cores=NC, num_subcores=NS)

  @functools.partial(
      pl.kernel,
      mesh=mesh,
      out_type=jax.ShapeDtypeStruct((tok, H), jnp.float32),
      scratch_types=[
          pltpu.VMEM((nchunk, chunk), jnp.int32),
          pltpu.VMEM((chunk, H), jnp.float32),
          pltpu.VMEM((chunk, H), jnp.float32),
          pltpu.SemaphoreType.DMA,
          pltpu.SemaphoreType.DMA,
          pltpu.SemaphoreType.DMA,
          pltpu.SemaphoreType.DMA,
      ],
  )
  def sc_gather(word_hbm, ids_hbm, out_hbm, idx_v, r0, r1, g0, g1, o0, o1):
    wid = lax.axis_index("s") * NC + lax.axis_index("c")
    base = wid * tpw
    bufs = (r0, r1)
    gsems = (g0, g1)
    osems = (o0, o1)
    pltpu.sync_copy(ids_hbm.at[wid], idx_v)

    gh = [None] * nchunk
    sh = [None] * nchunk
    gh[0] = pltpu.async_copy(word_hbm.at[idx_v.at[0]], bufs[0], gsems[0])
    for c in range(nchunk):
      b = c & 1
      gh[c].wait()
      if c + 1 < nchunk:
        nb = (c + 1) & 1
        if c >= 1:
          sh[c - 1].wait()
        gh[c + 1] = pltpu.async_copy(
            word_hbm.at[idx_v.at[c + 1]], bufs[nb], gsems[nb])
      sh[c] = pltpu.async_copy(
          bufs[b], out_hbm.at[pl.ds(base + c * chunk, chunk)], osems[b])
    if nchunk >= 2:
      sh[nchunk - 2].wait()
    sh[nchunk - 1].wait()

  return sc_gather


def _tc_ln_kernel(g_ref, pos_ref, tt_ref, type_ref, scale_ref, bias_ref,
                  out_ref, *rest):
  x = g_ref[...].reshape(SEQ_BLK, S, H)
  pos = pos_ref[...]
  tt = tt_ref[...]
  t0 = type_ref[0, :]
  t1 = type_ref[1, :]
  typ = jnp.where(tt == 1.0, t1[None, None, :], t0[None, None, :])
  h = x + pos[None] + typ
  mean = jnp.mean(h, axis=-1, keepdims=True)
  var = jnp.mean(h * h, axis=-1, keepdims=True) - mean * mean
  normed = (h - mean) * lax.rsqrt(var + EPS)
  out_ref[...] = normed * scale_ref[...] + bias_ref[...]


def _tc_ln_first_kernel(g_ref, pos_ref, tt_ref, type_ref, scale_ref,
                        bias_ref, out_ref):
  _tc_ln_kernel(g_ref, pos_ref, tt_ref, type_ref, scale_ref, bias_ref,
                out_ref)


def _tc_ln_acc_kernel(g_ref, pos_ref, tt_ref, type_ref, scale_ref,
                      bias_ref, o_prev_ref, out_ref):
  _tc_ln_kernel(g_ref, pos_ref, tt_ref, type_ref, scale_ref, bias_ref,
                out_ref)


@functools.lru_cache(maxsize=None)
def _tc_ln_call(seq_off, nseq, first):
  blk_off = seq_off // SEQ_BLK
  in_specs = [
      pl.BlockSpec((SEQ_BLK * S, H), lambda b: (b, 0)),
      pl.BlockSpec((S, H), lambda b: (0, 0)),
      pl.BlockSpec((SEQ_BLK, S, 1), lambda b: (b, 0, 0)),
      pl.BlockSpec((2, H), lambda b: (0, 0)),
      pl.BlockSpec((1, H), lambda b: (0, 0)),
      pl.BlockSpec((1, H), lambda b: (0, 0)),
  ]
  kwargs = {}
  if first:
    body = _tc_ln_first_kernel
  else:
    body = _tc_ln_acc_kernel
    in_specs = in_specs + [pl.BlockSpec(memory_space=pltpu.MemorySpace.HBM)]
    kwargs["input_output_aliases"] = {6: 0}
  return pl.pallas_call(
      body,
      grid=(nseq // SEQ_BLK,),
      in_specs=in_specs,
      out_specs=pl.BlockSpec((SEQ_BLK, S, H), lambda b: (blk_off + b, 0, 0)),
      out_shape=jax.ShapeDtypeStruct((B, S, H), jnp.float32),
      **kwargs,
  )


def kernel(input_ids, token_type_ids, position_ids, attention_mask,
           word_emb, pos_emb, type_emb, ln_scale, ln_bias):
  del position_ids, attention_mask
  ids_flat = input_ids.astype(jnp.int32).reshape(-1)
  tt_all = token_type_ids.astype(jnp.float32).reshape(B, S, 1)
  scale2 = ln_scale.reshape(1, H)
  bias2 = ln_bias.reshape(1, H)

  gathered = []
  off = 0
  for nseq in PIPE_SEQS:
    tok = nseq * S
    tpw = tok // NW
    chunk = min(CHUNK, tpw)
    ids_k = lax.dynamic_slice(ids_flat, (off * S,), (tok,)).reshape(
        NW, tpw // chunk, chunk)
    gathered.append(_sc_gather_fn(nseq)(word_emb, ids_k))
    off += nseq

  out = None
  off = 0
  for k, nseq in enumerate(PIPE_SEQS):
    tt_k = lax.dynamic_slice(tt_all, (off, 0, 0), (nseq, S, 1))
    args = (gathered[k], pos_emb, tt_k, type_emb, scale2, bias2)
    if out is None:
      out = _tc_ln_call(off, nseq, True)(*args)
    else:
      out = _tc_ln_call(off, nseq, False)(*args, out)
    off += nseq
  return out


# speedup vs baseline: 1.0299x; 1.0299x over previous
import functools

import jax
import jax.numpy as jnp
from jax import lax
from jax.experimental import pallas as pl
from jax.experimental.pallas import tpu as pltpu
from jax.experimental.pallas import tpu_sc as plsc

B, S, H = 64, 512, 768
V = 30522
EPS = 1e-12

NC, NS = 2, 16
NW = NC * NS
TOK = B * S
CHUNK = 64

PIPE_SEQS = (16, 16, 16, 16)
assert sum(PIPE_SEQS) == B

SEQ_BLK = 4


@functools.lru_cache(maxsize=None)
def _sc_gather_fn(nseq):
  tok = nseq * S
  tpw = tok // NW
  chunk = min(CHUNK, tpw)
  nchunk = tpw // chunk
  mesh = plsc.VectorSubcoreMesh(core_axis_name="c", subcore_axis_name="s",
                                num_cores=NC, num_subcores=NS)

  @functools.partial(
      pl.kernel,
      mesh=mesh,
      out_type=jax.ShapeDtypeStruct((tok, H), jnp.float32),
      scratch_types=[
          pltpu.VMEM((nchunk, chunk), jnp.int32),
          pltpu.VMEM((chunk, H), jnp.float32),
          pltpu.VMEM((chunk, H), jnp.float32),
          pltpu.SemaphoreType.DMA,
          pltpu.SemaphoreType.DMA,
          pltpu.SemaphoreType.DMA,
          pltpu.SemaphoreType.DMA,
      ],
  )
  def sc_gather(word_hbm, ids_hbm, out_hbm, idx_v, r0, r1, g0, g1, o0, o1):
    wid = lax.axis_index("s") * NC + lax.axis_index("c")
    base = wid * tpw
    bufs = (r0, r1)
    gsems = (g0, g1)
    osems = (o0, o1)
    pltpu.sync_copy(ids_hbm.at[wid], idx_v)

    gh = [None] * nchunk
    sh = [None] * nchunk
    gh[0] = pltpu.async_copy(word_hbm.at[idx_v.at[0]], bufs[0], gsems[0])
    for c in range(nchunk):
      b = c & 1
      gh[c].wait()
      if c + 1 < nchunk:
        nb = (c + 1) & 1
        if c >= 1:
          sh[c - 1].wait()
        gh[c + 1] = pltpu.async_copy(
            word_hbm.at[idx_v.at[c + 1]], bufs[nb], gsems[nb])
      sh[c] = pltpu.async_copy(
          bufs[b], out_hbm.at[pl.ds(base + c * chunk, chunk)], osems[b])
    if nchunk >= 2:
      sh[nchunk - 2].wait()
    sh[nchunk - 1].wait()

  return sc_gather


def _tc_ln_kernel(g_ref, pos_ref, tt_ref, type_ref, scale_ref, bias_ref,
                  out_ref, *rest):
  x = g_ref[...].reshape(SEQ_BLK, S, H)
  pos = pos_ref[...]
  tt = tt_ref[...]
  t0 = type_ref[0, :]
  t1 = type_ref[1, :]
  typ = jnp.where(tt == 1.0, t1[None, None, :], t0[None, None, :])
  h = x + pos[None] + typ
  mean = jnp.mean(h, axis=-1, keepdims=True)
  var = jnp.mean(h * h, axis=-1, keepdims=True) - mean * mean
  normed = (h - mean) * lax.rsqrt(var + EPS)
  out_ref[...] = normed * scale_ref[...] + bias_ref[...]


def _tc_ln_first_kernel(g_ref, pos_ref, tt_ref, type_ref, scale_ref,
                        bias_ref, out_ref):
  _tc_ln_kernel(g_ref, pos_ref, tt_ref, type_ref, scale_ref, bias_ref,
                out_ref)


def _tc_ln_acc_kernel(g_ref, pos_ref, tt_ref, type_ref, scale_ref,
                      bias_ref, o_prev_ref, out_ref):
  _tc_ln_kernel(g_ref, pos_ref, tt_ref, type_ref, scale_ref, bias_ref,
                out_ref)


@functools.lru_cache(maxsize=None)
def _tc_ln_call(seq_off, nseq, first):
  blk_off = seq_off // SEQ_BLK
  in_specs = [
      pl.BlockSpec((SEQ_BLK * S, H), lambda b: (b, 0)),
      pl.BlockSpec((S, H), lambda b: (0, 0)),
      pl.BlockSpec((SEQ_BLK, S, 1), lambda b: (blk_off + b, 0, 0)),
      pl.BlockSpec((2, H), lambda b: (0, 0)),
      pl.BlockSpec((1, H), lambda b: (0, 0)),
      pl.BlockSpec((1, H), lambda b: (0, 0)),
  ]
  kwargs = {}
  if first:
    body = _tc_ln_first_kernel
  else:
    body = _tc_ln_acc_kernel
    in_specs = in_specs + [pl.BlockSpec(memory_space=pltpu.MemorySpace.HBM)]
    kwargs["input_output_aliases"] = {6: 0}
  return pl.pallas_call(
      body,
      grid=(nseq // SEQ_BLK,),
      in_specs=in_specs,
      out_specs=pl.BlockSpec((SEQ_BLK, S, H), lambda b: (blk_off + b, 0, 0)),
      out_shape=jax.ShapeDtypeStruct((B, S, H), jnp.float32),
      **kwargs,
  )


def kernel(input_ids, token_type_ids, position_ids, attention_mask,
           word_emb, pos_emb, type_emb, ln_scale, ln_bias):
  del position_ids, attention_mask
  ids_flat = input_ids.astype(jnp.int32).reshape(-1)
  tt_all = token_type_ids.astype(jnp.float32).reshape(B, S, 1)
  scale2 = ln_scale.reshape(1, H)
  bias2 = ln_bias.reshape(1, H)

  gathered = []
  off = 0
  for nseq in PIPE_SEQS:
    tok = nseq * S
    tpw = tok // NW
    chunk = min(CHUNK, tpw)
    ids_k = lax.dynamic_slice(ids_flat, (off * S,), (tok,)).reshape(
        NW, tpw // chunk, chunk)
    gathered.append(_sc_gather_fn(nseq)(word_emb, ids_k))
    off += nseq

  out = None
  off = 0
  for k, nseq in enumerate(PIPE_SEQS):
    args = (gathered[k], pos_emb, tt_all, type_emb, scale2, bias2)
    if out is None:
      out = _tc_ln_call(off, nseq, True)(*args)
    else:
      out = _tc_ln_call(off, nseq, False)(*args, out)
    off += nseq
  return out
